# baseline (device time: 87095 ns/iter reference)
import jax
import jax.numpy as jnp
from jax import lax
from jax.experimental import pallas as pl
from jax.experimental.pallas import tpu as pltpu

N_DEV = 4
K = 8


def kernel(x):
    m, n = x.shape
    m_sub = m // K

    def body(
        x_ref,
        out_ref,
        comm_l,
        comm_r,
        stage_r,
        stage_l,
        send_r_sems,
        send_l_sems,
        recv_l_sems,
        recv_r_sems,
    ):
        my_x = lax.axis_index("x")
        my_y = lax.axis_index("y")
        my_z = lax.axis_index("z")
        is_first = my_y == 0
        is_last = my_y == N_DEV - 1
        is_mid = jnp.logical_and(my_y > 0, my_y < N_DEV - 1)
        left = jnp.maximum(my_y - 1, 0)
        right = jnp.minimum(my_y + 1, N_DEV - 1)

        def sub(ref, t):
            return ref.at[pl.ds(t * m_sub, m_sub), :]

        def send_right(t, src_ref):
            return pltpu.make_async_remote_copy(
                src_ref=src_ref,
                dst_ref=comm_l.at[t],
                send_sem=send_r_sems.at[t],
                recv_sem=recv_l_sems.at[t],
                device_id=(my_x, right, my_z),
                device_id_type=pl.DeviceIdType.MESH,
            )

        def send_left(t, src_ref):
            return pltpu.make_async_remote_copy(
                src_ref=src_ref,
                dst_ref=comm_r.at[t],
                send_sem=send_l_sems.at[t],
                recv_sem=recv_r_sems.at[t],
                device_id=(my_x, left, my_z),
                device_id_type=pl.DeviceIdType.MESH,
            )

        barrier_sem = pltpu.get_barrier_semaphore()

        @pl.when(jnp.logical_not(is_first))
        def _():
            pl.semaphore_signal(
                barrier_sem,
                inc=1,
                device_id=(my_x, left, my_z),
                device_id_type=pl.DeviceIdType.MESH,
            )

        @pl.when(jnp.logical_not(is_last))
        def _():
            pl.semaphore_signal(
                barrier_sem,
                inc=1,
                device_id=(my_x, right, my_z),
                device_id_type=pl.DeviceIdType.MESH,
            )

        n_nbrs = jnp.logical_not(is_first).astype(jnp.int32) + jnp.logical_not(
            is_last
        ).astype(jnp.int32)
        pl.semaphore_wait(barrier_sem, n_nbrs)

        for t in range(K):

            @pl.when(is_first)
            def _(t=t):
                send_right(t, sub(x_ref, t)).start()

            @pl.when(is_last)
            def _(t=t):
                send_left(t, sub(x_ref, t)).start()

        for t in range(K):
            @pl.when(jnp.logical_not(is_first))
            def _(t=t):
                send_right(t, sub(x_ref, t)).wait_recv()

            @pl.when(is_mid)
            def _(t=t):
                stage_r[t] = comm_l[t] + x_ref[pl.ds(t * m_sub, m_sub), :]
                send_right(t, stage_r.at[t]).start()

            @pl.when(jnp.logical_not(is_last))
            def _(t=t):
                send_left(t, sub(x_ref, t)).wait_recv()

            @pl.when(is_mid)
            def _(t=t):
                stage_l[t] = comm_r[t] + x_ref[pl.ds(t * m_sub, m_sub), :]
                send_left(t, stage_l.at[t]).start()

            @pl.when(is_first)
            def _(t=t):
                out_ref[pl.ds(t * m_sub, m_sub), :] = (
                    x_ref[pl.ds(t * m_sub, m_sub), :] + comm_r[t]
                )

            @pl.when(is_last)
            def _(t=t):
                out_ref[pl.ds(t * m_sub, m_sub), :] = (
                    x_ref[pl.ds(t * m_sub, m_sub), :] + comm_l[t]
                )

            @pl.when(is_mid)
            def _(t=t):
                out_ref[pl.ds(t * m_sub, m_sub), :] = stage_r[t] + comm_r[t]

        for t in range(K):

            @pl.when(jnp.logical_not(is_last))
            def _(t=t):
                send_right(t, sub(x_ref, t)).wait_send()

            @pl.when(jnp.logical_not(is_first))
            def _(t=t):
                send_left(t, sub(x_ref, t)).wait_send()

    return pl.pallas_call(
        body,
        out_shape=jax.ShapeDtypeStruct((m, n), x.dtype),
        in_specs=[pl.BlockSpec(memory_space=pltpu.VMEM)],
        out_specs=pl.BlockSpec(memory_space=pltpu.VMEM),
        scratch_shapes=[
            pltpu.VMEM((K, m_sub, n), x.dtype),
            pltpu.VMEM((K, m_sub, n), x.dtype),
            pltpu.VMEM((K, m_sub, n), x.dtype),
            pltpu.VMEM((K, m_sub, n), x.dtype),
            pltpu.SemaphoreType.DMA((K,)),
            pltpu.SemaphoreType.DMA((K,)),
            pltpu.SemaphoreType.DMA((K,)),
            pltpu.SemaphoreType.DMA((K,)),
        ],
        compiler_params=pltpu.CompilerParams(collective_id=0),
    )(x)


# device time: 31350 ns/iter; 2.7781x vs baseline; 2.7781x over previous
import jax
import jax.numpy as jnp
from jax import lax
from jax.experimental import pallas as pl
from jax.experimental.pallas import tpu as pltpu

N_DEV = 4
K = 8
LAG = 4


def kernel(x):
    m, n = x.shape
    m_sub = m // K
    n_half = n // 2

    def body(
        x_ref,
        out_ref,
        comm_l,
        comm_r,
        stage_r,
        stage_l,
        send_r_sems,
        send_l_sems,
        recv_l_sems,
        recv_r_sems,
        xsend_sems,
        xrecv_sems,
    ):
        my_x = lax.axis_index("x")
        my_y = lax.axis_index("y")
        my_z = lax.axis_index("z")
        left = jnp.maximum(my_y - 1, 0)
        right = jnp.minimum(my_y + 1, N_DEV - 1)
        col0 = my_x * n_half

        def rows(t):
            return pl.ds(t * m_sub, m_sub)

        def cols():
            return pl.ds(col0, n_half)

        def x_sub(t):
            return x_ref[rows(t), cols()]

        def send_right(t, src_ref):
            return pltpu.make_async_remote_copy(
                src_ref=src_ref,
                dst_ref=comm_l.at[t],
                send_sem=send_r_sems.at[t],
                recv_sem=recv_l_sems.at[t],
                device_id=(my_x, right, my_z),
                device_id_type=pl.DeviceIdType.MESH,
            )

        def send_left(t, src_ref):
            return pltpu.make_async_remote_copy(
                src_ref=src_ref,
                dst_ref=comm_r.at[t],
                send_sem=send_l_sems.at[t],
                recv_sem=recv_r_sems.at[t],
                device_id=(my_x, left, my_z),
                device_id_type=pl.DeviceIdType.MESH,
            )

        def x_rdma(t):
            sl_r, sl_c = rows(t), cols()
            return pltpu.make_async_remote_copy(
                src_ref=out_ref.at[sl_r, sl_c],
                dst_ref=out_ref.at[sl_r, sl_c],
                send_sem=xsend_sems.at[t],
                recv_sem=xrecv_sems.at[t],
                device_id=(1 - my_x, my_y, my_z),
                device_id_type=pl.DeviceIdType.MESH,
            )

        def relay(dirn, t):
            if dirn == 0:
                send_right(t, stage_r.at[t]).wait_recv()
                stage_r[t] = comm_l[t] + x_sub(t)
                send_right(t, stage_r.at[t]).start()
            else:
                send_left(t, stage_l.at[t]).wait_recv()
                stage_l[t] = comm_r[t] + x_sub(t)
                send_left(t, stage_l.at[t]).start()

        def finish_mid(t):
            out_ref[rows(t), cols()] = stage_r[t] + comm_r[t]
            x_rdma(t).start()

        def emit_middle(lead):
            lag = 1 - lead
            for t in range(LAG):
                relay(lead, t)
            for t in range(LAG, K):
                relay(lead, t)
                relay(lag, t - LAG)
                finish_mid(t - LAG)
            for t in range(K - LAG, K):
                relay(lag, t)
                finish_mid(t)

        barrier_sem = pltpu.get_barrier_semaphore()

        @pl.when(my_y > 0)
        def _():
            pl.semaphore_signal(
                barrier_sem, inc=1,
                device_id=(my_x, left, my_z),
                device_id_type=pl.DeviceIdType.MESH,
            )

        @pl.when(my_y < N_DEV - 1)
        def _():
            pl.semaphore_signal(
                barrier_sem, inc=1,
                device_id=(my_x, right, my_z),
                device_id_type=pl.DeviceIdType.MESH,
            )

        pl.semaphore_signal(
            barrier_sem, inc=1,
            device_id=(1 - my_x, my_y, my_z),
            device_id_type=pl.DeviceIdType.MESH,
        )
        n_nbrs = (
            1
            + (my_y > 0).astype(jnp.int32)
            + (my_y < N_DEV - 1).astype(jnp.int32)
        )
        pl.semaphore_wait(barrier_sem, n_nbrs)

        @pl.when(my_y == 0)
        def _():
            for t in range(K):
                send_right(t, x_ref.at[rows(t), cols()]).start()
            for t in range(K):
                send_left(t, stage_l.at[t]).wait_recv()
                out_ref[rows(t), cols()] = x_sub(t) + comm_r[t]
                x_rdma(t).start()

        @pl.when(my_y == N_DEV - 1)
        def _():
            for t in range(K):
                send_left(t, x_ref.at[rows(t), cols()]).start()
            for t in range(K):
                send_right(t, stage_r.at[t]).wait_recv()
                out_ref[rows(t), cols()] = x_sub(t) + comm_l[t]
                x_rdma(t).start()

        @pl.when(my_y == 1)
        def _():
            emit_middle(lead=0)

        @pl.when(my_y == 2)
        def _():
            emit_middle(lead=1)

        for t in range(K):
            x_rdma(t).wait_recv()
        for t in range(K):
            x_rdma(t).wait_send()

        @pl.when(my_y == 0)
        def _():
            for t in range(K):
                send_right(t, x_ref.at[rows(t), cols()]).wait_send()

        @pl.when(my_y == N_DEV - 1)
        def _():
            for t in range(K):
                send_left(t, x_ref.at[rows(t), cols()]).wait_send()

        @pl.when(jnp.logical_and(my_y > 0, my_y < N_DEV - 1))
        def _():
            for t in range(K):
                send_right(t, stage_r.at[t]).wait_send()
                send_left(t, stage_l.at[t]).wait_send()

    return pl.pallas_call(
        body,
        out_shape=jax.ShapeDtypeStruct((m, n), x.dtype),
        in_specs=[pl.BlockSpec(memory_space=pltpu.VMEM)],
        out_specs=pl.BlockSpec(memory_space=pltpu.VMEM),
        scratch_shapes=[
            pltpu.VMEM((K, m_sub, n_half), x.dtype),
            pltpu.VMEM((K, m_sub, n_half), x.dtype),
            pltpu.VMEM((K, m_sub, n_half), x.dtype),
            pltpu.VMEM((K, m_sub, n_half), x.dtype),
            pltpu.SemaphoreType.DMA((K,)),
            pltpu.SemaphoreType.DMA((K,)),
            pltpu.SemaphoreType.DMA((K,)),
            pltpu.SemaphoreType.DMA((K,)),
            pltpu.SemaphoreType.DMA((K,)),
            pltpu.SemaphoreType.DMA((K,)),
        ],
        compiler_params=pltpu.CompilerParams(collective_id=0),
    )(x)


# device time: 29400 ns/iter; 2.9624x vs baseline; 1.0663x over previous
import jax
import jax.numpy as jnp
from jax import lax
from jax.experimental import pallas as pl
from jax.experimental.pallas import tpu as pltpu

N_DEV = 4
K = 8


def kernel(x):
    m, n = x.shape
    m_sub = m // K
    n_q = n // 4

    def body(
        x_ref,
        out_ref,
        comm_l,
        comm_r,
        stage_r,
        stage_l,
        send_r_sems,
        send_l_sems,
        recv_l_sems,
        recv_r_sems,
        xsend_sems,
        xrecv_sems,
        zsend_sems,
        zrecv_sems,
        zxsend_sems,
        zxrecv_sems,
    ):
        my_x = lax.axis_index("x")
        my_y = lax.axis_index("y")
        my_z = lax.axis_index("z")
        left = jnp.maximum(my_y - 1, 0)
        right = jnp.minimum(my_y + 1, N_DEV - 1)
        zp = my_z % 2
        z_partner = my_z + 1 - 2 * zp
        col0 = (2 * my_x + zp) * n_q

        def rows(t):
            return pl.ds(t * m_sub, m_sub)

        def cols():
            return pl.ds(col0, n_q)

        def x_sub(t):
            return x_ref[rows(t), cols()]

        def send_right(t, src_ref):
            return pltpu.make_async_remote_copy(
                src_ref=src_ref,
                dst_ref=comm_l.at[t],
                send_sem=send_r_sems.at[t],
                recv_sem=recv_l_sems.at[t],
                device_id=(my_x, right, my_z),
                device_id_type=pl.DeviceIdType.MESH,
            )

        def send_left(t, src_ref):
            return pltpu.make_async_remote_copy(
                src_ref=src_ref,
                dst_ref=comm_r.at[t],
                send_sem=send_l_sems.at[t],
                recv_sem=recv_r_sems.at[t],
                device_id=(my_x, left, my_z),
                device_id_type=pl.DeviceIdType.MESH,
            )

        def xchg_rdma(t, sl_c, send_sems, recv_sems, dev_id):
            sl_r = rows(t)
            return pltpu.make_async_remote_copy(
                src_ref=out_ref.at[sl_r, sl_c],
                dst_ref=out_ref.at[sl_r, sl_c],
                send_sem=send_sems.at[t],
                recv_sem=recv_sems.at[t],
                device_id=dev_id,
                device_id_type=pl.DeviceIdType.MESH,
            )

        col_x = (2 * (1 - my_x) + zp) * n_q

        def x_rdma(t):
            return xchg_rdma(
                t, cols(), xsend_sems, xrecv_sems, (1 - my_x, my_y, my_z)
            )

        def z_rdma(t):
            return xchg_rdma(
                t, cols(), zsend_sems, zrecv_sems, (my_x, my_y, z_partner)
            )

        def zx_rdma(t):
            return xchg_rdma(
                t,
                pl.ds(col_x, n_q),
                zxsend_sems,
                zxrecv_sems,
                (my_x, my_y, z_partner),
            )

        def push_out(t):
            x_rdma(t).start()
            z_rdma(t).start()

        def relay(dirn, t):
            if dirn == 0:
                send_right(t, stage_r.at[t]).wait_recv()
                stage_r[t] = comm_l[t] + x_sub(t)
                send_right(t, stage_r.at[t]).start()
            else:
                send_left(t, stage_l.at[t]).wait_recv()
                stage_l[t] = comm_r[t] + x_sub(t)
                send_left(t, stage_l.at[t]).start()

        def emit_middle(lead):
            for t in range(K):
                relay(lead, t)
            for t in range(K):
                relay(1 - lead, t)
                out_ref[rows(t), cols()] = stage_r[t] + comm_r[t]
                push_out(t)

        barrier_sem = pltpu.get_barrier_semaphore()

        @pl.when(my_y > 0)
        def _():
            pl.semaphore_signal(
                barrier_sem, inc=1,
                device_id=(my_x, left, my_z),
                device_id_type=pl.DeviceIdType.MESH,
            )

        @pl.when(my_y < N_DEV - 1)
        def _():
            pl.semaphore_signal(
                barrier_sem, inc=1,
                device_id=(my_x, right, my_z),
                device_id_type=pl.DeviceIdType.MESH,
            )

        for dev_id in (
            (1 - my_x, my_y, my_z),
            (my_x, my_y, z_partner),
        ):
            pl.semaphore_signal(
                barrier_sem, inc=1,
                device_id=dev_id,
                device_id_type=pl.DeviceIdType.MESH,
            )
        n_nbrs = (
            2
            + (my_y > 0).astype(jnp.int32)
            + (my_y < N_DEV - 1).astype(jnp.int32)
        )
        pl.semaphore_wait(barrier_sem, n_nbrs)

        @pl.when(my_y == 0)
        def _():
            for t in range(K):
                send_right(t, x_ref.at[rows(t), cols()]).start()
            for t in range(K):
                send_left(t, stage_l.at[t]).wait_recv()
                out_ref[rows(t), cols()] = x_sub(t) + comm_r[t]
                push_out(t)

        @pl.when(my_y == N_DEV - 1)
        def _():
            for t in range(K):
                send_left(t, x_ref.at[rows(t), cols()]).start()
            for t in range(K):
                send_right(t, stage_r.at[t]).wait_recv()
                out_ref[rows(t), cols()] = x_sub(t) + comm_l[t]
                push_out(t)

        @pl.when(my_y == 1)
        def _():
            emit_middle(lead=0)

        @pl.when(my_y == 2)
        def _():
            emit_middle(lead=1)

        for t in range(K):
            x_rdma(t).wait_recv()
            zx_rdma(t).start()
        for t in range(K):
            z_rdma(t).wait_recv()
            zx_rdma(t).wait_recv()
        for t in range(K):
            x_rdma(t).wait_send()
            z_rdma(t).wait_send()
            zx_rdma(t).wait_send()

        @pl.when(my_y == 0)
        def _():
            for t in range(K):
                send_right(t, x_ref.at[rows(t), cols()]).wait_send()

        @pl.when(my_y == N_DEV - 1)
        def _():
            for t in range(K):
                send_left(t, x_ref.at[rows(t), cols()]).wait_send()

        @pl.when(jnp.logical_and(my_y > 0, my_y < N_DEV - 1))
        def _():
            for t in range(K):
                send_right(t, stage_r.at[t]).wait_send()
                send_left(t, stage_l.at[t]).wait_send()

    return pl.pallas_call(
        body,
        out_shape=jax.ShapeDtypeStruct((m, n), x.dtype),
        in_specs=[pl.BlockSpec(memory_space=pltpu.VMEM)],
        out_specs=pl.BlockSpec(memory_space=pltpu.VMEM),
        scratch_shapes=[
            pltpu.VMEM((K, m_sub, n_q), x.dtype),
            pltpu.VMEM((K, m_sub, n_q), x.dtype),
            pltpu.VMEM((K, m_sub, n_q), x.dtype),
            pltpu.VMEM((K, m_sub, n_q), x.dtype),
            pltpu.SemaphoreType.DMA((K,)),
            pltpu.SemaphoreType.DMA((K,)),
            pltpu.SemaphoreType.DMA((K,)),
            pltpu.SemaphoreType.DMA((K,)),
            pltpu.SemaphoreType.DMA((K,)),
            pltpu.SemaphoreType.DMA((K,)),
            pltpu.SemaphoreType.DMA((K,)),
            pltpu.SemaphoreType.DMA((K,)),
            pltpu.SemaphoreType.DMA((K,)),
            pltpu.SemaphoreType.DMA((K,)),
        ],
        compiler_params=pltpu.CompilerParams(collective_id=0),
    )(x)
